# baseline (speedup 1.0000x reference)
import functools

import jax
import jax.numpy as jnp
import numpy as np
from jax import lax
from jax.experimental import pallas as pl
from jax.experimental.pallas import tpu as pltpu
from jax.experimental.pallas import tpu_sc as plsc

def _rotl32(x, r):
  return ((x << np.uint32(r)) | (x >> np.uint32(32 - r))).astype(np.uint32)


def _threefry2x32_np(k0, k1, x0, x1):
  ks0, ks1 = np.uint32(k0), np.uint32(k1)
  ks2 = np.uint32(ks0 ^ ks1 ^ np.uint32(0x1BD11BDA))
  x0 = (x0 + ks0).astype(np.uint32)
  x1 = (x1 + ks1).astype(np.uint32)
  rot0, rot1 = (13, 15, 26, 6), (17, 29, 16, 24)
  inject = [(ks1, ks2, 1), (ks2, ks0, 2), (ks0, ks1, 3),
            (ks1, ks2, 4), (ks2, ks0, 5)]
  for blk in range(5):
    for r in rot0 if blk % 2 == 0 else rot1:
      x0 = (x0 + x1).astype(np.uint32)
      x1 = _rotl32(x1, r)
      x1 = (x1 ^ x0).astype(np.uint32)
    a, bb, c = inject[blk]
    x0 = (x0 + a).astype(np.uint32)
    x1 = (x1 + bb + np.uint32(c)).astype(np.uint32)
  return x0, x1


def _threefry_bits32_np(k0, k1, size):
  idx = np.arange(size, dtype=np.uint64)
  hi = (idx >> np.uint64(32)).astype(np.uint32)
  lo = (idx & np.uint64(0xFFFFFFFF)).astype(np.uint32)
  o0, o1 = _threefry2x32_np(k0, k1, hi, lo)
  return o0 ^ o1


def _np_randint(seed, shape, minval, maxval):
  k0, k1 = np.uint32(seed >> 32), np.uint32(seed & 0xFFFFFFFF)
  s0, s1 = _threefry2x32_np(k0, k1, np.zeros(2, np.uint32),
                            np.arange(2, dtype=np.uint32))
  size = int(np.prod(shape))
  higher = _threefry_bits32_np(s0[0], s1[0], size)
  lower = _threefry_bits32_np(s0[1], s1[1], size)
  span = np.uint32(maxval - minval)
  m0 = (2 ** 16) % int(span)
  mult = np.uint32(((m0 * m0) % (2 ** 32)) % int(span))
  off = ((higher % span) * mult + lower % span) % span
  return (np.int32(minval) + off.astype(np.int32)).reshape(shape)


_NUM_NEG = 1000
_PAD = 1008
_VREGS = _PAD // 16
_LANES = 16
_NC, _NS = 2, 16
_NW = _NC * _NS
_TC_ROWS = 256
_WIN = 6016
_NWIN = 15
_TAIL0 = _NWIN * _WIN
_GRP = 8


def _sc_group_exp_sums(logits2d, colh, posloc, poswin, row_base, n_rows,
                       p_seg):
  assert n_rows == _NW * _GRP
  per_w = _NWIN * _GRP * p_seg
  mesh = plsc.VectorSubcoreMesh(core_axis_name="c", subcore_axis_name="s")

  @functools.partial(
      pl.kernel,
      mesh=mesh,
      compiler_params=pltpu.CompilerParams(needs_layout_passes=False),
      out_type=[jax.ShapeDtypeStruct((_NW * _LANES,), jnp.float32),
                jax.ShapeDtypeStruct((_NW * _GRP * _LANES,), jnp.float32)],
      scratch_types=[
          pltpu.VMEM((per_w,), jnp.int32),
          pltpu.VMEM((_LANES,), jnp.int32),
          pltpu.VMEM((_LANES,), jnp.int32),
          pltpu.VMEM((_GRP, _WIN), jnp.float32),
          pltpu.VMEM((_GRP, _WIN), jnp.float32),
          pltpu.VMEM((_LANES,), jnp.float32),
          pltpu.VMEM((_GRP * _LANES,), jnp.float32),
          pltpu.SemaphoreType.DMA,
          pltpu.SemaphoreType.DMA,
      ],
  )
  def k(tab_hbm, col_hbm, posl_hbm, posw_hbm, pos_out, neg_out,
        colv, poslv, poswv, buf0, buf1, posb, negb, sem0, sem1):
    wid = lax.axis_index("s") * _NC + lax.axis_index("c")
    grow = row_base + wid * _GRP
    pltpu.sync_copy(col_hbm.at[pl.ds(wid * per_w, per_w)], colv)
    pltpu.sync_copy(posl_hbm.at[pl.ds(wid * _LANES, _LANES)], poslv)
    pltpu.sync_copy(posw_hbm.at[pl.ds(wid * _LANES, _LANES)], poswv)
    lane = lax.iota(jnp.int32, _LANES)
    pl_vec = poslv[...]
    pw_vec = poswv[...]
    sub_vec = jnp.where(lane < _GRP, lane, 0)
    bufs = (buf0, buf1)
    sems = (sem0, sem1)

    def start(i):
      return pltpu.async_copy(
          tab_hbm.at[pl.ds(grow, _GRP), pl.ds(i * _WIN, _WIN)],
          bufs[i % 2], sems[i % 2])

    posacc = jnp.zeros((_LANES,), jnp.float32)
    zero16 = jnp.zeros((_LANES,), jnp.float32)
    for s in range(_GRP):
      negb[pl.ds(s * _LANES, _LANES)] = zero16
    d = start(0)
    for i in range(_NWIN):
      d.wait()
      if i + 1 < _NWIN:
        d = start(i + 1)
      buf = bufs[i % 2]
      for s in range(_GRP):
        base = (i * _GRP + s) * p_seg
        svec = jnp.full((_LANES,), s, jnp.int32)

        def body(j, acc, base=base, buf=buf, svec=svec):
          lv = colv[pl.ds(base + j * 16, 16)]
          valid = lv >= 0
          idx = jnp.where(valid, lv, 0)
          g = plsc.load_gather(buf, [svec, idx])
          return acc + jnp.where(valid, jnp.exp(g), 0.0)

        acc = lax.fori_loop(0, p_seg // 16, body,
                            jnp.zeros((_LANES,), jnp.float32))
        negb[pl.ds(s * _LANES, _LANES)] = (
            negb[pl.ds(s * _LANES, _LANES)]
            + jnp.where(lane == i, jnp.sum(acc), 0.0))
      pvalid = (pw_vec == i) & (lane < _GRP)
      pidx = jnp.where(pvalid, pl_vec, 0)
      pg = plsc.load_gather(buf, [sub_vec, pidx])
      posacc = posacc + jnp.where(pvalid, jnp.exp(pg), 0.0)
    posb[...] = posacc
    pltpu.sync_copy(posb, pos_out.at[pl.ds(wid * _LANES, _LANES)])
    pltpu.sync_copy(
        negb, neg_out.at[pl.ds(wid * _GRP * _LANES, _GRP * _LANES)])

  pos_p, neg_p = k(logits2d, colh, posloc, poswin)
  pos = pos_p.reshape(_NW, _LANES)[:, :_GRP].reshape(-1)
  neg = neg_p.reshape(_NW, _GRP, _LANES).sum(axis=2).reshape(-1)
  return pos, neg


def _tc_row_exp_sums(lmat, m8, tcol, nrows, width):
  blk = 32

  def body(l_ref, m_ref, t_ref, p_ref, n_ref):
    ev = jnp.exp(l_ref[...])
    m = m_ref[...].astype(jnp.float32)
    colid = lax.broadcasted_iota(jnp.int32, (blk, width), 1)
    t = t_ref[...][:, :1]
    pos_row = jnp.sum(jnp.where(colid == t, ev, 0.0), axis=1)
    neg_row = jnp.sum(ev * m, axis=1)
    p_ref[...] = jnp.broadcast_to(pos_row[:, None], (blk, 128))
    n_ref[...] = jnp.broadcast_to(neg_row[:, None], (blk, 128))

  pos, neg = pl.pallas_call(
      body,
      grid=(nrows // blk,),
      in_specs=[
          pl.BlockSpec((blk, width), lambda i: (i, 0)),
          pl.BlockSpec((blk, width), lambda i: (i, 0)),
          pl.BlockSpec((blk, 128), lambda i: (i, 0)),
      ],
      out_specs=[
          pl.BlockSpec((blk, 128), lambda i: (i, 0)),
          pl.BlockSpec((blk, 128), lambda i: (i, 0)),
      ],
      out_shape=[jax.ShapeDtypeStruct((nrows, 128), jnp.float32),
                 jax.ShapeDtypeStruct((nrows, 128), jnp.float32)],
  )(lmat, m8, tcol)
  return pos[:, 0], neg[:, 0]


def _tc_loss(pos_exp, neg_sum, mask_flat):
  n = pos_exp.shape[0]
  n_pad = -n % 1024
  pos_p = jnp.concatenate(
      [pos_exp, jnp.ones((n_pad,), jnp.float32)]).reshape(-1, 128)
  neg_p = jnp.concatenate(
      [neg_sum, jnp.zeros((n_pad,), jnp.float32)]).reshape(-1, 128)
  m_p = jnp.concatenate(
      [mask_flat, jnp.zeros((n_pad,), jnp.float32)]).reshape(-1, 128)

  def body(p_ref, n_ref, m_ref, o_ref):
    p = p_ref[...]
    ng = n_ref[...]
    m = m_ref[...]
    loss = -jnp.log(p / (p + ng + 1e-08))
    val = jnp.sum(loss * m) / (jnp.sum(m) + 1e-08)
    o_ref[...] = jnp.full((1, 1), val, jnp.float32)

  out = pl.pallas_call(
      body,
      out_shape=jax.ShapeDtypeStruct((1, 1), jnp.float32),
  )(pos_p, neg_p, m_p)
  return out[0, 0]


def kernel(logits, targets, mask):
  b, s, v = logits.shape
  n = b * s
  logits2d = logits.reshape(n, v)
  t_flat = targets.reshape(-1).astype(jnp.int32)
  neg_np = _np_randint(1234, (n, _NUM_NEG), 0, v)

  m8 = np.zeros((_TC_ROWS, v), np.int8)
  np.add.at(
      m8,
      (np.repeat(np.arange(_TC_ROWS), _NUM_NEG), neg_np[:_TC_ROWS].ravel()),
      1)
  tcol = jnp.broadcast_to(t_flat[:_TC_ROWS, None], (_TC_ROWS, 128))

  n_sc = n - _TC_ROWS
  neg_sc_np = neg_np[_TC_ROWS:]
  win_np = neg_sc_np // _WIN
  loc_np = neg_sc_np - win_np * _WIN
  counts = np.zeros((n_sc, _NWIN), np.int64)
  for r in range(n_sc):
    counts[r] = np.bincount(win_np[r], minlength=_NWIN + 1)[:_NWIN]
  p_seg = -(-int(counts.max()) // 16) * 16
  colh = np.full((_NW, _NWIN, _GRP, p_seg), -1, np.int32)
  for r in range(n_sc):
    w, s = r // _GRP, r % _GRP
    for i in range(_NWIN):
      sel = win_np[r] == i
      c = loc_np[r][sel]
      colh[w, i, s, :len(c)] = c
  tailw = v - _TAIL0
  m8t = np.zeros((n_sc, tailw), np.int8)
  for r in range(n_sc):
    tc_ = neg_sc_np[r][neg_sc_np[r] >= _TAIL0] - _TAIL0
    np.add.at(m8t[r], tc_, 1)
  t_sc = t_flat[_TC_ROWS:]
  posloc = (t_sc % _WIN).reshape(_NW, _GRP)
  poswin = (t_sc // _WIN).reshape(_NW, _GRP)
  pad8 = jnp.zeros((_NW, _LANES - _GRP), jnp.int32)
  posloc = jnp.concatenate([posloc, pad8], axis=1).reshape(-1)
  poswin = jnp.concatenate([poswin, pad8 - 1], axis=1).reshape(-1)

  pos_sc, neg_sc = _sc_group_exp_sums(
      logits2d, jnp.asarray(colh.reshape(-1)), posloc, poswin,
      _TC_ROWS, n_sc, p_seg)
  pos_tc, neg_tc = _tc_row_exp_sums(
      logits2d, jnp.asarray(m8), tcol, _TC_ROWS, v)
  lt = logits2d[_TC_ROWS:, _TAIL0:]
  tcol_t = jnp.broadcast_to((t_sc - _TAIL0)[:, None], (n_sc, 128))
  pos_t, neg_t = _tc_row_exp_sums(
      lt, jnp.asarray(m8t), tcol_t, n_sc, tailw)
  pos_sc = pos_sc + pos_t
  neg_sc = neg_sc + neg_t

  pos_e = jnp.concatenate([pos_tc, pos_sc])
  neg_e = jnp.concatenate([neg_tc, neg_sc])
  return _tc_loss(pos_e, neg_e, mask.reshape(-1).astype(jnp.float32))


# speedup vs baseline: 1.1540x; 1.1540x over previous
import functools

import jax
import jax.numpy as jnp
import numpy as np
from jax import lax
from jax.experimental import pallas as pl
from jax.experimental.pallas import tpu as pltpu
from jax.experimental.pallas import tpu_sc as plsc

def _rotl32(x, r):
  return ((x << np.uint32(r)) | (x >> np.uint32(32 - r))).astype(np.uint32)


def _threefry2x32_np(k0, k1, x0, x1):
  ks0, ks1 = np.uint32(k0), np.uint32(k1)
  ks2 = np.uint32(ks0 ^ ks1 ^ np.uint32(0x1BD11BDA))
  x0 = (x0 + ks0).astype(np.uint32)
  x1 = (x1 + ks1).astype(np.uint32)
  rot0, rot1 = (13, 15, 26, 6), (17, 29, 16, 24)
  inject = [(ks1, ks2, 1), (ks2, ks0, 2), (ks0, ks1, 3),
            (ks1, ks2, 4), (ks2, ks0, 5)]
  for blk in range(5):
    for r in rot0 if blk % 2 == 0 else rot1:
      x0 = (x0 + x1).astype(np.uint32)
      x1 = _rotl32(x1, r)
      x1 = (x1 ^ x0).astype(np.uint32)
    a, bb, c = inject[blk]
    x0 = (x0 + a).astype(np.uint32)
    x1 = (x1 + bb + np.uint32(c)).astype(np.uint32)
  return x0, x1


def _threefry_bits32_np(k0, k1, size):
  idx = np.arange(size, dtype=np.uint64)
  hi = (idx >> np.uint64(32)).astype(np.uint32)
  lo = (idx & np.uint64(0xFFFFFFFF)).astype(np.uint32)
  o0, o1 = _threefry2x32_np(k0, k1, hi, lo)
  return o0 ^ o1


def _np_randint(seed, shape, minval, maxval):
  k0, k1 = np.uint32(seed >> 32), np.uint32(seed & 0xFFFFFFFF)
  s0, s1 = _threefry2x32_np(k0, k1, np.zeros(2, np.uint32),
                            np.arange(2, dtype=np.uint32))
  size = int(np.prod(shape))
  higher = _threefry_bits32_np(s0[0], s1[0], size)
  lower = _threefry_bits32_np(s0[1], s1[1], size)
  span = np.uint32(maxval - minval)
  m0 = (2 ** 16) % int(span)
  mult = np.uint32(((m0 * m0) % (2 ** 32)) % int(span))
  off = ((higher % span) * mult + lower % span) % span
  return (np.int32(minval) + off.astype(np.int32)).reshape(shape)


_NUM_NEG = 1000
_PAD = 1008
_VREGS = _PAD // 16
_LANES = 16
_NC, _NS = 2, 16
_NW = _NC * _NS
_TC_ROWS = 256


def _sc_row_exp_sums(logits2d, cols_flat, row_base, n_rows, vocab):
  rows_per_w = n_rows // _NW
  per_w = rows_per_w * _PAD
  mesh = plsc.VectorSubcoreMesh(core_axis_name="c", subcore_axis_name="s")

  @functools.partial(
      pl.kernel,
      mesh=mesh,
      compiler_params=pltpu.CompilerParams(needs_layout_passes=False),
      out_type=[jax.ShapeDtypeStruct((_NW * _LANES,), jnp.float32),
                jax.ShapeDtypeStruct((_NW * _LANES,), jnp.float32)],
      scratch_types=[
          pltpu.VMEM((per_w,), jnp.int32),
          pltpu.VMEM((vocab,), jnp.float32),
          pltpu.VMEM((_LANES,), jnp.float32),
          pltpu.VMEM((_LANES,), jnp.float32),
      ],
  )
  def k(tab_hbm, col_hbm, pos_out, neg_out, colv, rowbuf, posb, negb):
    wid = lax.axis_index("s") * _NC + lax.axis_index("c")
    base = wid * per_w
    row0 = wid * rows_per_w
    pltpu.sync_copy(col_hbm.at[pl.ds(base, per_w)], colv)
    lane = lax.iota(jnp.int32, _LANES)
    posacc = jnp.zeros((_LANES,), jnp.float32)
    negacc = jnp.zeros((_LANES,), jnp.float32)
    for r in range(rows_per_w):
      rb = r * _PAD
      pltpu.sync_copy(tab_hbm.at[row_base + row0 + r], rowbuf)

      def body(j, acc, rb=rb):
        cols = colv[pl.ds(rb + j * 16, 16)]
        vals = plsc.load_gather(rowbuf, [cols])
        return acc + jnp.exp(vals)

      acc = lax.fori_loop(0, _VREGS - 1, body,
                          jnp.zeros((_LANES,), jnp.float32))
      cols = colv[pl.ds(rb + (_VREGS - 1) * 16, 16)]
      e = jnp.exp(plsc.load_gather(rowbuf, [cols]))
      acc = acc + jnp.where(lane < 8, e, 0.0)
      pos_s = jnp.sum(jnp.where(lane == 8, e, 0.0))
      neg_s = jnp.sum(acc)
      sel = lane == r
      posacc = jnp.where(sel, pos_s, posacc)
      negacc = jnp.where(sel, neg_s, negacc)
    posb[...] = posacc
    negb[...] = negacc
    pltpu.sync_copy(posb, pos_out.at[pl.ds(wid * _LANES, _LANES)])
    pltpu.sync_copy(negb, neg_out.at[pl.ds(wid * _LANES, _LANES)])

  pos_p, neg_p = k(logits2d, cols_flat)
  pos = pos_p.reshape(_NW, _LANES)[:, :rows_per_w].reshape(-1)
  neg = neg_p.reshape(_NW, _LANES)[:, :rows_per_w].reshape(-1)
  return pos, neg


def _tc_row_exp_sums(logits2d, m8, tcol, vocab):
  blk = 32

  def body(l_ref, m_ref, t_ref, p_ref, n_ref):
    ev = jnp.exp(l_ref[...])
    m = m_ref[...].astype(jnp.float32)
    colid = lax.broadcasted_iota(jnp.int32, (blk, vocab), 1)
    t = t_ref[...][:, :1]
    pos_row = jnp.sum(jnp.where(colid == t, ev, 0.0), axis=1)
    neg_row = jnp.sum(ev * m, axis=1)
    p_ref[...] = jnp.broadcast_to(pos_row[:, None], (blk, 128))
    n_ref[...] = jnp.broadcast_to(neg_row[:, None], (blk, 128))

  pos, neg = pl.pallas_call(
      body,
      grid=(_TC_ROWS // blk,),
      in_specs=[
          pl.BlockSpec((blk, vocab), lambda i: (i, 0)),
          pl.BlockSpec((blk, vocab), lambda i: (i, 0)),
          pl.BlockSpec((blk, 128), lambda i: (i, 0)),
      ],
      out_specs=[
          pl.BlockSpec((blk, 128), lambda i: (i, 0)),
          pl.BlockSpec((blk, 128), lambda i: (i, 0)),
      ],
      out_shape=[jax.ShapeDtypeStruct((_TC_ROWS, 128), jnp.float32),
                 jax.ShapeDtypeStruct((_TC_ROWS, 128), jnp.float32)],
  )(logits2d, m8, tcol)
  return pos[:, 0], neg[:, 0]


def _tc_loss(pos_exp, neg_sum, mask_flat):
  n = pos_exp.shape[0]
  n_pad = -n % 1024
  pos_p = jnp.concatenate(
      [pos_exp, jnp.ones((n_pad,), jnp.float32)]).reshape(-1, 128)
  neg_p = jnp.concatenate(
      [neg_sum, jnp.zeros((n_pad,), jnp.float32)]).reshape(-1, 128)
  m_p = jnp.concatenate(
      [mask_flat, jnp.zeros((n_pad,), jnp.float32)]).reshape(-1, 128)

  def body(p_ref, n_ref, m_ref, o_ref):
    p = p_ref[...]
    ng = n_ref[...]
    m = m_ref[...]
    loss = -jnp.log(p / (p + ng + 1e-08))
    val = jnp.sum(loss * m) / (jnp.sum(m) + 1e-08)
    o_ref[...] = jnp.full((1, 1), val, jnp.float32)

  out = pl.pallas_call(
      body,
      out_shape=jax.ShapeDtypeStruct((1, 1), jnp.float32),
  )(pos_p, neg_p, m_p)
  return out[0, 0]


def kernel(logits, targets, mask):
  b, s, v = logits.shape
  n = b * s
  logits2d = logits.reshape(n, v)
  t_flat = targets.reshape(-1).astype(jnp.int32)
  neg_np = _np_randint(1234, (n, _NUM_NEG), 0, v)

  m8 = np.zeros((_TC_ROWS, v), np.int8)
  np.add.at(
      m8,
      (np.repeat(np.arange(_TC_ROWS), _NUM_NEG), neg_np[:_TC_ROWS].ravel()),
      1)
  tcol = jnp.broadcast_to(t_flat[:_TC_ROWS, None], (_TC_ROWS, 128))

  n_sc = n - _TC_ROWS
  cols_sc = jnp.concatenate(
      [jnp.asarray(neg_np[_TC_ROWS:]),
       jnp.broadcast_to(t_flat[_TC_ROWS:, None], (n_sc, _PAD - _NUM_NEG))],
      axis=1)

  pos_sc, neg_sc = _sc_row_exp_sums(
      logits2d, cols_sc.reshape(-1), _TC_ROWS, n_sc, v)
  pos_tc, neg_tc = _tc_row_exp_sums(logits2d, jnp.asarray(m8), tcol, v)

  pos_e = jnp.concatenate([pos_tc, pos_sc])
  neg_e = jnp.concatenate([neg_tc, neg_sc])
  return _tc_loss(pos_e, neg_e, mask.reshape(-1).astype(jnp.float32))


# speedup vs baseline: 1.1886x; 1.0300x over previous
import functools

import jax
import jax.numpy as jnp
import numpy as np
from jax import lax
from jax.experimental import pallas as pl
from jax.experimental.pallas import tpu as pltpu
from jax.experimental.pallas import tpu_sc as plsc

def _rotl32(x, r):
  return ((x << np.uint32(r)) | (x >> np.uint32(32 - r))).astype(np.uint32)


def _threefry2x32_np(k0, k1, x0, x1):
  ks0, ks1 = np.uint32(k0), np.uint32(k1)
  ks2 = np.uint32(ks0 ^ ks1 ^ np.uint32(0x1BD11BDA))
  x0 = (x0 + ks0).astype(np.uint32)
  x1 = (x1 + ks1).astype(np.uint32)
  rot0, rot1 = (13, 15, 26, 6), (17, 29, 16, 24)
  inject = [(ks1, ks2, 1), (ks2, ks0, 2), (ks0, ks1, 3),
            (ks1, ks2, 4), (ks2, ks0, 5)]
  for blk in range(5):
    for r in rot0 if blk % 2 == 0 else rot1:
      x0 = (x0 + x1).astype(np.uint32)
      x1 = _rotl32(x1, r)
      x1 = (x1 ^ x0).astype(np.uint32)
    a, bb, c = inject[blk]
    x0 = (x0 + a).astype(np.uint32)
    x1 = (x1 + bb + np.uint32(c)).astype(np.uint32)
  return x0, x1


def _threefry_bits32_np(k0, k1, size):
  idx = np.arange(size, dtype=np.uint64)
  hi = (idx >> np.uint64(32)).astype(np.uint32)
  lo = (idx & np.uint64(0xFFFFFFFF)).astype(np.uint32)
  o0, o1 = _threefry2x32_np(k0, k1, hi, lo)
  return o0 ^ o1


def _np_randint(seed, shape, minval, maxval):
  k0, k1 = np.uint32(seed >> 32), np.uint32(seed & 0xFFFFFFFF)
  s0, s1 = _threefry2x32_np(k0, k1, np.zeros(2, np.uint32),
                            np.arange(2, dtype=np.uint32))
  size = int(np.prod(shape))
  higher = _threefry_bits32_np(s0[0], s1[0], size)
  lower = _threefry_bits32_np(s0[1], s1[1], size)
  span = np.uint32(maxval - minval)
  m0 = (2 ** 16) % int(span)
  mult = np.uint32(((m0 * m0) % (2 ** 32)) % int(span))
  off = ((higher % span) * mult + lower % span) % span
  return (np.int32(minval) + off.astype(np.int32)).reshape(shape)


_NUM_NEG = 1000
_PAD = 1008
_VREGS = _PAD // 16
_LANES = 16
_NC, _NS = 2, 16
_NW = _NC * _NS
_TC_ROWS = 224


def _sc_row_exp_sums(logits2d, cols_flat, row_base, n_rows, vocab):
  rows_per_w = n_rows // _NW
  per_w = rows_per_w * _PAD
  mesh = plsc.VectorSubcoreMesh(core_axis_name="c", subcore_axis_name="s")

  @functools.partial(
      pl.kernel,
      mesh=mesh,
      compiler_params=pltpu.CompilerParams(needs_layout_passes=False),
      out_type=[jax.ShapeDtypeStruct((_NW * _LANES,), jnp.float32),
                jax.ShapeDtypeStruct((_NW * _LANES,), jnp.float32)],
      scratch_types=[
          pltpu.VMEM((per_w,), jnp.int32),
          pltpu.VMEM((vocab,), jnp.float32),
          pltpu.VMEM((_LANES,), jnp.float32),
          pltpu.VMEM((_LANES,), jnp.float32),
      ],
  )
  def k(tab_hbm, col_hbm, pos_out, neg_out, colv, rowbuf, posb, negb):
    wid = lax.axis_index("s") * _NC + lax.axis_index("c")
    base = wid * per_w
    row0 = wid * rows_per_w
    pltpu.sync_copy(col_hbm.at[pl.ds(base, per_w)], colv)
    lane = lax.iota(jnp.int32, _LANES)
    posacc = jnp.zeros((_LANES,), jnp.float32)
    negacc = jnp.zeros((_LANES,), jnp.float32)
    for r in range(rows_per_w):
      rb = r * _PAD
      pltpu.sync_copy(tab_hbm.at[row_base + row0 + r], rowbuf)

      def body(j, acc, rb=rb):
        cols = colv[pl.ds(rb + j * 16, 16)]
        vals = plsc.load_gather(rowbuf, [cols])
        return acc + jnp.exp(vals)

      acc = lax.fori_loop(0, _VREGS - 1, body,
                          jnp.zeros((_LANES,), jnp.float32))
      cols = colv[pl.ds(rb + (_VREGS - 1) * 16, 16)]
      e = jnp.exp(plsc.load_gather(rowbuf, [cols]))
      acc = acc + jnp.where(lane < 8, e, 0.0)
      pos_s = jnp.sum(jnp.where(lane == 8, e, 0.0))
      neg_s = jnp.sum(acc)
      sel = lane == r
      posacc = jnp.where(sel, pos_s, posacc)
      negacc = jnp.where(sel, neg_s, negacc)
    posb[...] = posacc
    negb[...] = negacc
    pltpu.sync_copy(posb, pos_out.at[pl.ds(wid * _LANES, _LANES)])
    pltpu.sync_copy(negb, neg_out.at[pl.ds(wid * _LANES, _LANES)])

  pos_p, neg_p = k(logits2d, cols_flat)
  pos = pos_p.reshape(_NW, _LANES)[:, :rows_per_w].reshape(-1)
  neg = neg_p.reshape(_NW, _LANES)[:, :rows_per_w].reshape(-1)
  return pos, neg


def _tc_row_exp_sums(logits2d, m8, tcol, vocab):
  blk = 32

  def body(l_ref, m_ref, t_ref, p_ref, n_ref):
    ev = jnp.exp(l_ref[...])
    m = m_ref[...].astype(jnp.float32)
    colid = lax.broadcasted_iota(jnp.int32, (blk, vocab), 1)
    t = t_ref[...][:, :1]
    pos_row = jnp.sum(jnp.where(colid == t, ev, 0.0), axis=1)
    neg_row = jnp.sum(ev * m, axis=1)
    p_ref[...] = jnp.broadcast_to(pos_row[:, None], (blk, 128))
    n_ref[...] = jnp.broadcast_to(neg_row[:, None], (blk, 128))

  pos, neg = pl.pallas_call(
      body,
      grid=(_TC_ROWS // blk,),
      in_specs=[
          pl.BlockSpec((blk, vocab), lambda i: (i, 0)),
          pl.BlockSpec((blk, vocab), lambda i: (i, 0)),
          pl.BlockSpec((blk, 128), lambda i: (i, 0)),
      ],
      out_specs=[
          pl.BlockSpec((blk, 128), lambda i: (i, 0)),
          pl.BlockSpec((blk, 128), lambda i: (i, 0)),
      ],
      out_shape=[jax.ShapeDtypeStruct((_TC_ROWS, 128), jnp.float32),
                 jax.ShapeDtypeStruct((_TC_ROWS, 128), jnp.float32)],
  )(logits2d, m8, tcol)
  return pos[:, 0], neg[:, 0]


def _tc_loss(pos_exp, neg_sum, mask_flat):
  n = pos_exp.shape[0]
  n_pad = -n % 1024
  pos_p = jnp.concatenate(
      [pos_exp, jnp.ones((n_pad,), jnp.float32)]).reshape(-1, 128)
  neg_p = jnp.concatenate(
      [neg_sum, jnp.zeros((n_pad,), jnp.float32)]).reshape(-1, 128)
  m_p = jnp.concatenate(
      [mask_flat, jnp.zeros((n_pad,), jnp.float32)]).reshape(-1, 128)

  def body(p_ref, n_ref, m_ref, o_ref):
    p = p_ref[...]
    ng = n_ref[...]
    m = m_ref[...]
    loss = -jnp.log(p / (p + ng + 1e-08))
    val = jnp.sum(loss * m) / (jnp.sum(m) + 1e-08)
    o_ref[...] = jnp.full((1, 1), val, jnp.float32)

  out = pl.pallas_call(
      body,
      out_shape=jax.ShapeDtypeStruct((1, 1), jnp.float32),
  )(pos_p, neg_p, m_p)
  return out[0, 0]


def kernel(logits, targets, mask):
  b, s, v = logits.shape
  n = b * s
  logits2d = logits.reshape(n, v)
  t_flat = targets.reshape(-1).astype(jnp.int32)
  neg_np = _np_randint(1234, (n, _NUM_NEG), 0, v)

  m8 = np.zeros((_TC_ROWS, v), np.int8)
  np.add.at(
      m8,
      (np.repeat(np.arange(_TC_ROWS), _NUM_NEG), neg_np[:_TC_ROWS].ravel()),
      1)
  tcol = jnp.broadcast_to(t_flat[:_TC_ROWS, None], (_TC_ROWS, 128))

  n_sc = n - _TC_ROWS
  cols_sc = jnp.concatenate(
      [jnp.asarray(neg_np[_TC_ROWS:]),
       jnp.broadcast_to(t_flat[_TC_ROWS:, None], (n_sc, _PAD - _NUM_NEG))],
      axis=1)

  pos_sc, neg_sc = _sc_row_exp_sums(
      logits2d, cols_sc.reshape(-1), _TC_ROWS, n_sc, v)
  pos_tc, neg_tc = _tc_row_exp_sums(logits2d, jnp.asarray(m8), tcol, v)

  pos_e = jnp.concatenate([pos_tc, pos_sc])
  neg_e = jnp.concatenate([neg_tc, neg_sc])
  return _tc_loss(pos_e, neg_e, mask.reshape(-1).astype(jnp.float32))
